# baseline (speedup 1.0000x reference)
import functools

import jax
import jax.numpy as jnp
from jax import lax
from jax.experimental import pallas as pl
from jax.experimental.pallas import tpu as pltpu
from jax.experimental.pallas import tpu_sc as plsc

B = 4096
K = 64
NT = 1_000_000
POOLW = 2000
NC = 2
NS = 16
NW = NC * NS
RPW = B // NW
SSTRIDE = 32
SN = NT // SSTRIDE
SSTEPS = 15
FSTEPS = 6
TAILM = (1 << 21) - 1


def _body(h_hbm, r_hbm, t_hbm, rnd_hbm, ed_hbm, dl_hbm, pool_hbm, tab_hbm,
          samp_hbm,
          negt_hbm, mask_hbm,
          h_v, r_v, t_v, dom_v, len_v, rand_v, flat_v, negt_v, mask_v,
          lo_v, hi_v, midb_v, midb2_v, gth_v, gtl_v, idx16_v, tail16_v, cnt16_v,
          spl_v, sem, sem_s, sem_c):
    wid = (lax.axis_index("s").astype(jnp.int32) * jnp.int32(NC)
           + lax.axis_index("c").astype(jnp.int32))
    base = wid * jnp.int32(RPW)

    pass

    ds_s0 = pltpu.async_copy(samp_hbm, spl_v, sem_s)

    pltpu.sync_copy(h_hbm.at[pl.ds(base, RPW)], h_v)
    pltpu.sync_copy(r_hbm.at[pl.ds(base, RPW)], r_v)
    pltpu.sync_copy(t_hbm.at[pl.ds(base, RPW)], t_v)
    pltpu.sync_copy(rnd_hbm.at[:, pl.ds(base, RPW)], rand_v)

    pltpu.async_copy(ed_hbm.at[t_v], dom_v, sem).wait()
    pltpu.async_copy(dl_hbm.at[dom_v], len_v, sem).wait()

    @pl.loop(jnp.int32(0), jnp.int32(K))
    def _(k):
        for g in range(8):
            sl = pl.ds(g * 16, 16)
            lf = len_v[sl].astype(jnp.float32)
            iv = (rand_v[k, sl] * lf).astype(jnp.int32)
            iv = jnp.minimum(iv, len_v[sl] - 1)
            flat_v[k, sl] = dom_v[sl] * POOLW + iv
            mask_v[k, sl] = jnp.ones((16,), jnp.int32)

    cand_descs = [pltpu.async_copy(pool_hbm.at[flat_v.at[jnp.int32(k)]],
                                   negt_v.at[jnp.int32(k)], sem_c)
                  for k in range(K)]

    ds_s0.wait()
    for s in range(2):
        for g in range(8):
            sl = pl.ds(g * 16, 16)
            kh = h_v[sl] << 10
            kl = (r_v[sl] + s) << 21

            @pl.loop(jnp.int32(0), jnp.int32(SSTEPS),
                     init_carry=(jnp.zeros((16,), jnp.int32),
                                 jnp.full((16,), SN, jnp.int32)))
            def _(step, carry, kh=kh, kl=kl):
                lo, hi = carry
                mid = (lo + hi) >> 1
                smid = jnp.minimum(mid, SN - 1)
                gh = plsc.load_gather(spl_v, [2 * smid + 1])
                gl = plsc.load_gather(spl_v, [2 * smid])
                upd = lo < hi
                ge = (gh > kh) | ((gh == kh) & (gl >= kl))
                return (jnp.where(upd & ~ge, mid + 1, lo),
                        jnp.where(upd & ge, mid, hi))

            ps, _unused = _
            row = jnp.maximum(ps - 1, 0)
            lo_v[s, sl] = row * SSTRIDE
            hi_v[s, sl] = ps * SSTRIDE

    @pl.loop(jnp.int32(0), jnp.int32(FSTEPS))
    def _(step):
        for s in range(2):
            for g in range(8):
                sl = pl.ds(g * 16, 16)
                mid = (lo_v[s, sl] + hi_v[s, sl]) >> 1
                w = 2 * jnp.minimum(mid, NT - 1)
                midb_v[s, sl] = w
                midb2_v[s, sl] = w + 1
        i0, i1 = jnp.int32(0), jnp.int32(1)
        d0 = pltpu.async_copy(tab_hbm.at[midb2_v.at[i0]], gth_v.at[i0], sem)
        d1 = pltpu.async_copy(tab_hbm.at[midb2_v.at[i1]], gth_v.at[i1], sem)
        d2 = pltpu.async_copy(tab_hbm.at[midb_v.at[i0]], gtl_v.at[i0], sem)
        d3 = pltpu.async_copy(tab_hbm.at[midb_v.at[i1]], gtl_v.at[i1], sem)
        d0.wait(); d1.wait(); d2.wait(); d3.wait()
        for s in range(2):
            for g in range(8):
                sl = pl.ds(g * 16, 16)
                lo = lo_v[s, sl]
                hi = hi_v[s, sl]
                mid = (lo + hi) >> 1
                kh = h_v[sl] << 10
                kl = (r_v[sl] + s) << 21
                gh = gth_v[s, sl]
                gl = gtl_v[s, sl]
                upd = lo < hi
                ge = (gh > kh) | ((gh == kh) & (gl >= kl))
                lo_v[s, sl] = jnp.where(upd & ~ge, mid + 1, lo)
                hi_v[s, sl] = jnp.where(upd & ge, mid, hi)

    for d in cand_descs:
        d.wait()

    for g in range(8):
        sl = pl.ds(g * 16, 16)
        p1 = lo_v[0, sl]
        p2 = lo_v[1, sl]

        cnt = p2 - p1
        for s in (8, 4, 2, 1):
            cnt16_v[...] = cnt
            perm = lax.iota(jnp.int32, 16) ^ s
            cnt = jnp.maximum(cnt, plsc.load_gather(cnt16_v, [perm]))
        maxc = cnt[0]

        @pl.loop(jnp.int32(0), maxc)
        def _(j, p1=p1, p2=p2, sl=sl):
            pos = p1 + j
            valid = pos < p2
            idx16_v[...] = 2 * jnp.where(valid, pos, 0)
            pltpu.async_copy(tab_hbm.at[idx16_v], tail16_v, sem).wait()
            tail = jnp.where(valid, tail16_v[...] & TAILM, -1)
            for k in range(K):
                m = mask_v[k, sl]
                mask_v[k, sl] = jnp.where(negt_v[k, sl] == tail, 0, m)

    pltpu.sync_copy(negt_v, negt_hbm.at[:, pl.ds(base, RPW)])
    pltpu.sync_copy(mask_v, mask_hbm.at[:, pl.ds(base, RPW)])


@jax.jit
def _sampler(h32, r32, t32, randT, ed, dl, pool_flat, table64, samp64):
    return _launch(h32, r32, t32, randT, ed, dl, pool_flat, table64, samp64)


def _launch(h32, r32, t32, randT, ed, dl, pool_flat, table64, samp64):
    mesh = plsc.VectorSubcoreMesh(core_axis_name="c", subcore_axis_name="s",
                                  num_cores=NC, num_subcores=NS)
    launch = pl.kernel(
        _body,
        out_type=(jax.ShapeDtypeStruct((K, B), jnp.int32),
                  jax.ShapeDtypeStruct((K, B), jnp.int32)),
        mesh=mesh,
        compiler_params=pltpu.CompilerParams(needs_layout_passes=False),
        scratch_types=(
            pltpu.VMEM((RPW,), jnp.int32),
            pltpu.VMEM((RPW,), jnp.int32),
            pltpu.VMEM((RPW,), jnp.int32),
            pltpu.VMEM((RPW,), jnp.int32),
            pltpu.VMEM((RPW,), jnp.int32),
            pltpu.VMEM((K, RPW), jnp.float32),
            pltpu.VMEM((K, RPW), jnp.int32),
            pltpu.VMEM((K, RPW), jnp.int32),
            pltpu.VMEM((K, RPW), jnp.int32),
            pltpu.VMEM((2, RPW), jnp.int32),
            pltpu.VMEM((2, RPW), jnp.int32),
            pltpu.VMEM((2, RPW), jnp.int32),
            pltpu.VMEM((2, RPW), jnp.int32),
            pltpu.VMEM((2, RPW), jnp.int32),
            pltpu.VMEM((2, RPW), jnp.int32),
            pltpu.VMEM((16,), jnp.int32),
            pltpu.VMEM((16,), jnp.int32),
            pltpu.VMEM((16,), jnp.int32),
            pltpu.VMEM((2 * SN,), jnp.int32),
            pltpu.SemaphoreType.DMA,
            pltpu.SemaphoreType.DMA,
            pltpu.SemaphoreType.DMA,
        ),
    )
    return launch(h32, r32, t32, randT, ed, dl, pool_flat, table64, samp64)


def kernel(positive_batch, domain_padded, domain_len, entity_domain, sorted_hashes, rand):
    h64 = positive_batch[:, 0]
    r64 = positive_batch[:, 1]
    t64 = positive_batch[:, 2]
    h32 = h64.astype(jnp.int32)
    r32 = r64.astype(jnp.int32)
    t32 = t64.astype(jnp.int32)
    randT = rand.T
    pool_flat = domain_padded.reshape(-1)
    pair = lax.bitcast_convert_type(sorted_hashes, jnp.int32).reshape(2 * NT)
    spair = lax.bitcast_convert_type(sorted_hashes[::SSTRIDE],
                                     jnp.int32).reshape(2 * SN)

    negt_T, mask_T = _sampler(h32, r32, t32, randT,
                              entity_domain.astype(jnp.int32),
                              domain_len.astype(jnp.int32),
                              pool_flat, pair, spair)

    neg_tails = negt_T.T.astype(jnp.int64)
    mask = mask_T.T.astype(jnp.bool_)
    neg_triples = jnp.stack([
        jnp.broadcast_to(h64[:, None], (B, K)),
        jnp.broadcast_to(r64[:, None], (B, K)),
        neg_tails,
    ], axis=-1)
    return neg_triples, mask


# speedup vs baseline: 7.3803x; 7.3803x over previous
import functools

import jax
import jax.numpy as jnp
from jax import lax
from jax.experimental import pallas as pl
from jax.experimental.pallas import tpu as pltpu
from jax.experimental.pallas import tpu_sc as plsc

B = 4096
K = 64
NT = 1_000_000
POOLW = 2000
NC = 2
NS = 16
NW = NC * NS
RPW = B // NW
SSTRIDE = 32
SN = NT // SSTRIDE
SSTEPS = 15
FSTEPS = 6
TAILM = (1 << 21) - 1


def _body(h_hbm, r_hbm, t_hbm, rnd_hbm, ed_hbm, dl_hbm, pool_hbm, th_hbm, tl_hbm,
          sth_hbm, stl_hbm,
          negt_hbm, mask_hbm,
          h_v, r_v, t_v, dom_v, len_v, rand_v, flat_v, negt_v, mask_v,
          lo_v, hi_v, midb_v, gth_v, gtl_v, idx16_v, tail16_v, cnt16_v,
          sth_v, stl_v, sem, sem_s, sem_c):
    wid = lax.axis_index("s") * NC + lax.axis_index("c")
    base = wid * RPW

    ds_s0 = pltpu.async_copy(sth_hbm, sth_v, sem_s)
    ds_s1 = pltpu.async_copy(stl_hbm, stl_v, sem_s)

    pltpu.sync_copy(h_hbm.at[pl.ds(base, RPW)], h_v)
    pltpu.sync_copy(r_hbm.at[pl.ds(base, RPW)], r_v)
    pltpu.sync_copy(t_hbm.at[pl.ds(base, RPW)], t_v)
    pltpu.sync_copy(rnd_hbm.at[:, pl.ds(base, RPW)], rand_v)

    pltpu.async_copy(ed_hbm.at[t_v], dom_v, sem).wait()
    pltpu.async_copy(dl_hbm.at[dom_v], len_v, sem).wait()

    @pl.loop(0, K)
    def _(k):
        for g in range(8):
            sl = pl.ds(g * 16, 16)
            lf = len_v[sl].astype(jnp.float32)
            iv = (rand_v[k, sl] * lf).astype(jnp.int32)
            iv = jnp.minimum(iv, len_v[sl] - 1)
            flat_v[k, sl] = dom_v[sl] * POOLW + iv
            mask_v[k, sl] = jnp.ones((16,), jnp.int32)

    cand_descs = [pltpu.async_copy(pool_hbm.at[flat_v.at[k]], negt_v.at[k], sem_c)
                  for k in range(K)]

    ds_s0.wait()
    ds_s1.wait()
    for s in range(2):
        for g in range(8):
            sl = pl.ds(g * 16, 16)
            kh = h_v[sl] << 10
            kl = (r_v[sl] + s) << 21

            @pl.loop(0, SSTEPS,
                     init_carry=(jnp.zeros((16,), jnp.int32),
                                 jnp.full((16,), SN, jnp.int32)))
            def _(step, carry, kh=kh, kl=kl):
                lo, hi = carry
                mid = (lo + hi) >> 1
                smid = jnp.minimum(mid, SN - 1)
                gh = plsc.load_gather(sth_v, [smid])
                gl = plsc.load_gather(stl_v, [smid])
                upd = lo < hi
                ge = (gh > kh) | ((gh == kh) & (gl >= kl))
                return (jnp.where(upd & ~ge, mid + 1, lo),
                        jnp.where(upd & ge, mid, hi))

            ps, _unused = _
            row = jnp.maximum(ps - 1, 0)
            lo_v[s, sl] = row * SSTRIDE
            hi_v[s, sl] = ps * SSTRIDE

    @pl.loop(0, FSTEPS)
    def _(step):
        for s in range(2):
            for g in range(8):
                sl = pl.ds(g * 16, 16)
                mid = (lo_v[s, sl] + hi_v[s, sl]) >> 1
                midb_v[s, sl] = jnp.minimum(mid, NT - 1)
        d0 = pltpu.async_copy(th_hbm.at[midb_v.at[0]], gth_v.at[0], sem)
        d1 = pltpu.async_copy(th_hbm.at[midb_v.at[1]], gth_v.at[1], sem)
        d2 = pltpu.async_copy(tl_hbm.at[midb_v.at[0]], gtl_v.at[0], sem)
        d3 = pltpu.async_copy(tl_hbm.at[midb_v.at[1]], gtl_v.at[1], sem)
        d0.wait(); d1.wait(); d2.wait(); d3.wait()
        for s in range(2):
            for g in range(8):
                sl = pl.ds(g * 16, 16)
                lo = lo_v[s, sl]
                hi = hi_v[s, sl]
                mid = (lo + hi) >> 1
                kh = h_v[sl] << 10
                kl = (r_v[sl] + s) << 21
                gh = gth_v[s, sl]
                gl = gtl_v[s, sl]
                upd = lo < hi
                ge = (gh > kh) | ((gh == kh) & (gl >= kl))
                lo_v[s, sl] = jnp.where(upd & ~ge, mid + 1, lo)
                hi_v[s, sl] = jnp.where(upd & ge, mid, hi)

    for d in cand_descs:
        d.wait()

    for g in range(8):
        sl = pl.ds(g * 16, 16)
        p1 = lo_v[0, sl]
        p2 = lo_v[1, sl]

        cnt = p2 - p1
        for s in (8, 4, 2, 1):
            cnt16_v[...] = cnt
            perm = lax.iota(jnp.int32, 16) ^ s
            cnt = jnp.maximum(cnt, plsc.load_gather(cnt16_v, [perm]))
        maxc = cnt[0]

        @pl.loop(0, maxc)
        def _(j, p1=p1, p2=p2, sl=sl):
            pos = p1 + j
            valid = pos < p2
            idx16_v[...] = jnp.where(valid, pos, 0)
            pltpu.async_copy(tl_hbm.at[idx16_v], tail16_v, sem).wait()
            tail = jnp.where(valid, tail16_v[...] & TAILM, -1)
            for k in range(K):
                m = mask_v[k, sl]
                mask_v[k, sl] = jnp.where(negt_v[k, sl] == tail, 0, m)

    pltpu.sync_copy(negt_v, negt_hbm.at[:, pl.ds(base, RPW)])
    pltpu.sync_copy(mask_v, mask_hbm.at[:, pl.ds(base, RPW)])


@jax.jit
def _sampler(h32, r32, t32, randT, ed, dl, pool_flat, th, tl, sth, stl):
    with jax.enable_x64(False):
        return _launch(h32, r32, t32, randT, ed, dl, pool_flat, th, tl, sth, stl)


def _launch(h32, r32, t32, randT, ed, dl, pool_flat, th, tl, sth, stl):
    mesh = plsc.VectorSubcoreMesh(core_axis_name="c", subcore_axis_name="s",
                                  num_cores=NC, num_subcores=NS)
    launch = pl.kernel(
        _body,
        out_type=(jax.ShapeDtypeStruct((K, B), jnp.int32),
                  jax.ShapeDtypeStruct((K, B), jnp.int32)),
        mesh=mesh,
        compiler_params=pltpu.CompilerParams(needs_layout_passes=False),
        scratch_types=(
            pltpu.VMEM((RPW,), jnp.int32),
            pltpu.VMEM((RPW,), jnp.int32),
            pltpu.VMEM((RPW,), jnp.int32),
            pltpu.VMEM((RPW,), jnp.int32),
            pltpu.VMEM((RPW,), jnp.int32),
            pltpu.VMEM((K, RPW), jnp.float32),
            pltpu.VMEM((K, RPW), jnp.int32),
            pltpu.VMEM((K, RPW), jnp.int32),
            pltpu.VMEM((K, RPW), jnp.int32),
            pltpu.VMEM((2, RPW), jnp.int32),
            pltpu.VMEM((2, RPW), jnp.int32),
            pltpu.VMEM((2, RPW), jnp.int32),
            pltpu.VMEM((2, RPW), jnp.int32),
            pltpu.VMEM((2, RPW), jnp.int32),
            pltpu.VMEM((16,), jnp.int32),
            pltpu.VMEM((16,), jnp.int32),
            pltpu.VMEM((16,), jnp.int32),
            pltpu.VMEM((SN,), jnp.int32),
            pltpu.VMEM((SN,), jnp.int32),
            pltpu.SemaphoreType.DMA,
            pltpu.SemaphoreType.DMA,
            pltpu.SemaphoreType.DMA,
        ),
    )
    return launch(h32, r32, t32, randT, ed, dl, pool_flat, th, tl, sth, stl)


def kernel(positive_batch, domain_padded, domain_len, entity_domain, sorted_hashes, rand):
    h64 = positive_batch[:, 0]
    r64 = positive_batch[:, 1]
    t64 = positive_batch[:, 2]
    h32 = h64.astype(jnp.int32)
    r32 = r64.astype(jnp.int32)
    t32 = t64.astype(jnp.int32)
    randT = rand.T
    pool_flat = domain_padded.reshape(-1)
    th = (sorted_hashes >> 32).astype(jnp.int32)
    tl = (sorted_hashes & 0xFFFFFFFF).astype(jnp.int32)
    sth = th[::SSTRIDE]
    stl = tl[::SSTRIDE]

    negt_T, mask_T = _sampler(h32, r32, t32, randT,
                              entity_domain.astype(jnp.int32),
                              domain_len.astype(jnp.int32),
                              pool_flat, th, tl, sth, stl)

    neg_tails = negt_T.T.astype(jnp.int64)
    mask = mask_T.T.astype(jnp.bool_)
    neg_triples = jnp.stack([
        jnp.broadcast_to(h64[:, None], (B, K)),
        jnp.broadcast_to(r64[:, None], (B, K)),
        neg_tails,
    ], axis=-1)
    return neg_triples, mask


# speedup vs baseline: 7.4399x; 1.0081x over previous
import functools

import jax
import jax.numpy as jnp
from jax import lax
from jax.experimental import pallas as pl
from jax.experimental.pallas import tpu as pltpu
from jax.experimental.pallas import tpu_sc as plsc

B = 4096
K = 64
NT = 1_000_000
POOLW = 2000
NC = 2
NS = 16
NW = NC * NS
RPW = B // NW
SSTRIDE = 32
SN = NT // SSTRIDE
SSTEPS = 15
FSTEPS = 6
TAILM = (1 << 21) - 1


def _body(h_hbm, r_hbm, t_hbm, rnd_hbm, ed_hbm, dl_hbm, pool_hbm, th_hbm, tl_hbm,
          sth_hbm, stl_hbm,
          negt_hbm, mask_hbm,
          h_v, r_v, t_v, dom_v, len_v, rand_v, flat_v, negt_v, mask_v,
          lo_v, hi_v, midb_v, gth_v, gtl_v, idx16_v, tail16_v, cnt16_v,
          sth_v, stl_v, sem, sem_s, sem_c):
    wid = lax.axis_index("s") * NC + lax.axis_index("c")
    base = wid * RPW

    ds_s0 = pltpu.async_copy(sth_hbm, sth_v, sem_s)
    ds_s1 = pltpu.async_copy(stl_hbm, stl_v, sem_s)

    pltpu.sync_copy(h_hbm.at[pl.ds(base, RPW)], h_v)
    pltpu.sync_copy(r_hbm.at[pl.ds(base, RPW)], r_v)
    pltpu.sync_copy(t_hbm.at[pl.ds(base, RPW)], t_v)
    pltpu.sync_copy(rnd_hbm.at[:, pl.ds(base, RPW)], rand_v)

    pltpu.async_copy(ed_hbm.at[t_v], dom_v, sem).wait()
    pltpu.async_copy(dl_hbm.at[dom_v], len_v, sem).wait()

    @pl.loop(0, K)
    def _(k):
        for g in range(8):
            sl = pl.ds(g * 16, 16)
            lf = len_v[sl].astype(jnp.float32)
            iv = (rand_v[k, sl] * lf).astype(jnp.int32)
            iv = jnp.minimum(iv, len_v[sl] - 1)
            flat_v[k, sl] = dom_v[sl] * POOLW + iv
            mask_v[k, sl] = jnp.ones((16,), jnp.int32)

    cand_descs = [pltpu.async_copy(pool_hbm.at[flat_v.at[k]], negt_v.at[k], sem_c)
                  for k in range(K)]

    ds_s0.wait()
    ds_s1.wait()
    for s in range(2):
        for g in range(8):
            sl = pl.ds(g * 16, 16)
            kh = h_v[sl] << 10
            kl = (r_v[sl] + s) << 21

            @pl.loop(0, SSTEPS,
                     init_carry=(jnp.zeros((16,), jnp.int32),
                                 jnp.full((16,), SN, jnp.int32)))
            def _(step, carry, kh=kh, kl=kl):
                lo, hi = carry
                mid = (lo + hi) >> 1
                smid = jnp.minimum(mid, SN - 1)
                gh = plsc.load_gather(sth_v, [smid])
                gl = plsc.load_gather(stl_v, [smid])
                upd = lo < hi
                ge = (gh > kh) | ((gh == kh) & (gl >= kl))
                return (jnp.where(upd & ~ge, mid + 1, lo),
                        jnp.where(upd & ge, mid, hi))

            ps, _unused = _
            row = jnp.maximum(ps - 1, 0)
            lo_v[s, sl] = row * SSTRIDE
            hi_v[s, sl] = ps * SSTRIDE

    @pl.loop(0, FSTEPS)
    def _(step):
        for s in range(2):
            for g in range(8):
                sl = pl.ds(g * 16, 16)
                mid = (lo_v[s, sl] + hi_v[s, sl]) >> 1
                midb_v[s, sl] = jnp.minimum(mid, NT - 1)
        d0 = pltpu.async_copy(th_hbm.at[midb_v.at[0]], gth_v.at[0], sem)
        d1 = pltpu.async_copy(th_hbm.at[midb_v.at[1]], gth_v.at[1], sem)
        d2 = pltpu.async_copy(tl_hbm.at[midb_v.at[0]], gtl_v.at[0], sem)
        d3 = pltpu.async_copy(tl_hbm.at[midb_v.at[1]], gtl_v.at[1], sem)
        d0.wait(); d1.wait(); d2.wait(); d3.wait()
        for s in range(2):
            for g in range(8):
                sl = pl.ds(g * 16, 16)
                lo = lo_v[s, sl]
                hi = hi_v[s, sl]
                mid = (lo + hi) >> 1
                kh = h_v[sl] << 10
                kl = (r_v[sl] + s) << 21
                gh = gth_v[s, sl]
                gl = gtl_v[s, sl]
                upd = lo < hi
                ge = (gh > kh) | ((gh == kh) & (gl >= kl))
                lo_v[s, sl] = jnp.where(upd & ~ge, mid + 1, lo)
                hi_v[s, sl] = jnp.where(upd & ge, mid, hi)

    for d in cand_descs:
        d.wait()

    for g in range(8):
        sl = pl.ds(g * 16, 16)
        p1 = lo_v[0, sl]
        p2 = lo_v[1, sl]

        cnt = p2 - p1
        for s in (8, 4, 2, 1):
            cnt16_v[...] = cnt
            perm = lax.iota(jnp.int32, 16) ^ s
            cnt = jnp.maximum(cnt, plsc.load_gather(cnt16_v, [perm]))
        maxc = cnt[0]

        @pl.loop(0, maxc)
        def _(j, p1=p1, p2=p2, sl=sl):
            pos = p1 + j
            valid = pos < p2
            idx16_v[...] = jnp.where(valid, pos, 0)
            pltpu.async_copy(tl_hbm.at[idx16_v], tail16_v, sem).wait()
            tail = jnp.where(valid, tail16_v[...] & TAILM, -1)
            for k in range(K):
                m = mask_v[k, sl]
                mask_v[k, sl] = jnp.where(negt_v[k, sl] == tail, 0, m)

    pltpu.sync_copy(negt_v, negt_hbm.at[:, pl.ds(base, RPW)])
    pltpu.sync_copy(mask_v, mask_hbm.at[:, pl.ds(base, RPW)])


@jax.jit
def _sampler(h32, r32, t32, randT, ed, dl, pool_flat, th, tl, sth, stl):
    with jax.enable_x64(False):
        return _launch(h32, r32, t32, randT, ed, dl, pool_flat, th, tl, sth, stl)


def _launch(h32, r32, t32, randT, ed, dl, pool_flat, th, tl, sth, stl):
    mesh = plsc.VectorSubcoreMesh(core_axis_name="c", subcore_axis_name="s",
                                  num_cores=NC, num_subcores=NS)
    launch = pl.kernel(
        _body,
        out_type=(jax.ShapeDtypeStruct((K, B), jnp.int32),
                  jax.ShapeDtypeStruct((K, B), jnp.int32)),
        mesh=mesh,
        compiler_params=pltpu.CompilerParams(needs_layout_passes=False),
        scratch_types=(
            pltpu.VMEM((RPW,), jnp.int32),
            pltpu.VMEM((RPW,), jnp.int32),
            pltpu.VMEM((RPW,), jnp.int32),
            pltpu.VMEM((RPW,), jnp.int32),
            pltpu.VMEM((RPW,), jnp.int32),
            pltpu.VMEM((K, RPW), jnp.float32),
            pltpu.VMEM((K, RPW), jnp.int32),
            pltpu.VMEM((K, RPW), jnp.int32),
            pltpu.VMEM((K, RPW), jnp.int32),
            pltpu.VMEM((2, RPW), jnp.int32),
            pltpu.VMEM((2, RPW), jnp.int32),
            pltpu.VMEM((2, RPW), jnp.int32),
            pltpu.VMEM((2, RPW), jnp.int32),
            pltpu.VMEM((2, RPW), jnp.int32),
            pltpu.VMEM((16,), jnp.int32),
            pltpu.VMEM((16,), jnp.int32),
            pltpu.VMEM((16,), jnp.int32),
            pltpu.VMEM((SN,), jnp.int32),
            pltpu.VMEM((SN,), jnp.int32),
            pltpu.SemaphoreType.DMA,
            pltpu.SemaphoreType.DMA,
            pltpu.SemaphoreType.DMA,
        ),
    )
    return launch(h32, r32, t32, randT, ed, dl, pool_flat, th, tl, sth, stl)


def kernel(positive_batch, domain_padded, domain_len, entity_domain, sorted_hashes, rand):
    h64 = positive_batch[:, 0]
    r64 = positive_batch[:, 1]
    t64 = positive_batch[:, 2]
    h32 = h64.astype(jnp.int32)
    r32 = r64.astype(jnp.int32)
    t32 = t64.astype(jnp.int32)
    randT = rand.T
    pool_flat = domain_padded.reshape(-1)
    th = (sorted_hashes >> 32).astype(jnp.int32)
    tl = (sorted_hashes & 0xFFFFFFFF).astype(jnp.int32)
    sth = th[::SSTRIDE]
    stl = tl[::SSTRIDE]

    negt_T, mask_T = _sampler(h32, r32, t32, randT,
                              entity_domain.astype(jnp.int32),
                              domain_len.astype(jnp.int32),
                              pool_flat, th, tl, sth, stl)

    mask = mask_T.T.astype(jnp.bool_)
    trip32 = jnp.stack([
        jnp.broadcast_to(h32[:, None], (B, K)),
        jnp.broadcast_to(r32[:, None], (B, K)),
        negt_T.T,
    ], axis=-1)
    neg_triples = trip32.astype(jnp.int64)
    return neg_triples, mask
